# baseline (speedup 1.0000x reference)
import jax
import jax.numpy as jnp
from jax.experimental import pallas as pl
from jax.experimental.pallas import tpu as pltpu


def _moe_kernel(x_ref, up_ref, down_ref, rw_ref, gw_ref, out_ref, ids_ref,
                up_bf, down_bf):
    E, H, F = up_ref.shape

    @pl.when(pl.program_id(0) == 0)
    def _pack_weights():
        for e in range(E):
            up_bf[:, e * F:(e + 1) * F] = up_ref[e].astype(jnp.bfloat16)
            down_bf[e * F:(e + 1) * F, :] = down_ref[e].astype(jnp.bfloat16)

    xb = x_ref[...]
    logits = jax.lax.dot_general(
        xb, rw_ref[...], (((1,), (1,)), ((), ())),
        preferred_element_type=jnp.float32)
    ids = jnp.argmax(logits, axis=-1).astype(jnp.int32)

    xbf = xb.astype(jnp.bfloat16)
    B = xb.shape[0]
    CH = (E // 2) * F
    expert_out = jnp.zeros((B, H), jnp.float32)
    for c in range(0, E * F, CH):
        up = jnp.dot(xbf, up_bf[:, c:c + CH],
                     preferred_element_type=jnp.float32)
        act = up * jax.nn.sigmoid(up)
        col_expert = (c + jax.lax.broadcasted_iota(jnp.int32, (B, CH), 1)) // F
        act = jnp.where(col_expert == ids[:, None], act, 0.0)
        expert_out = expert_out + jnp.dot(
            act.astype(jnp.bfloat16), down_bf[c:c + CH, :],
            preferred_element_type=jnp.float32)

    gate_logit = jax.lax.dot_general(
        xb, gw_ref[...], (((1,), (1,)), ((), ())),
        preferred_element_type=jnp.float32)
    gate = jax.nn.sigmoid(gate_logit)

    out_ref[...] = xb + gate * expert_out
    ids_ref[0, 0, :] = ids


def kernel(x, up_proj, down_proj, router_w, gate_w):
    N, H = x.shape
    E, _, F = up_proj.shape

    BLK = 512
    grid = N // BLK
    out, ids3 = pl.pallas_call(
        _moe_kernel,
        grid=(grid,),
        in_specs=[
            pl.BlockSpec((BLK, H), lambda i: (i, 0)),
            pl.BlockSpec((E, H, F), lambda i: (0, 0, 0)),
            pl.BlockSpec((E, F, H), lambda i: (0, 0, 0)),
            pl.BlockSpec((E, H), lambda i: (0, 0)),
            pl.BlockSpec((1, H), lambda i: (0, 0)),
        ],
        out_specs=[
            pl.BlockSpec((BLK, H), lambda i: (i, 0)),
            pl.BlockSpec((1, 1, BLK), lambda i: (i, 0, 0)),
        ],
        out_shape=[
            jax.ShapeDtypeStruct((N, H), jnp.float32),
            jax.ShapeDtypeStruct((grid, 1, BLK), jnp.int32),
        ],
        scratch_shapes=[
            pltpu.VMEM((H, E * F), jnp.bfloat16),
            pltpu.VMEM((E * F, H), jnp.bfloat16),
        ],
    )(x, up_proj, down_proj, router_w, gate_w)
    return out, ids3.reshape(N)


# speedup vs baseline: 1.0487x; 1.0487x over previous
import jax
import jax.numpy as jnp
from jax.experimental import pallas as pl
from jax.experimental.pallas import tpu as pltpu


def _moe_kernel(x_ref, up_ref, down_ref, rw_ref, gw_ref, out_ref, ids_ref,
                up_bf, down_bf):
    E, H, F = up_ref.shape

    @pl.when(pl.program_id(0) == 0)
    def _pack_weights():
        for e in range(E):
            up_bf[:, e * F:(e + 1) * F] = up_ref[e].astype(jnp.bfloat16)
            down_bf[e * F:(e + 1) * F, :] = down_ref[e].astype(jnp.bfloat16)

    xb = x_ref[...]
    logits = jax.lax.dot_general(
        xb, rw_ref[...], (((1,), (1,)), ((), ())),
        preferred_element_type=jnp.float32)
    ids = jnp.argmax(logits, axis=-1).astype(jnp.int32)

    gate_logit = jax.lax.dot_general(
        xb, gw_ref[...], (((1,), (1,)), ((), ())),
        preferred_element_type=jnp.float32)
    gate = jax.nn.sigmoid(gate_logit)

    rowmax = jnp.max(logits, axis=1, keepdims=True)
    eqs = logits == rowmax
    ce = eqs.astype(jnp.float32)
    zf = jnp.zeros_like(ce)
    for sh in (1, 2, 4):
        ce = ce + jnp.concatenate([zf[:, :sh], ce[:, :-sh]], axis=1)
    ohg = jnp.where(eqs & (ce == 1.0), gate, 0.0)

    xbf = xb.astype(jnp.bfloat16)
    up = jnp.dot(xbf, up_bf[...], preferred_element_type=jnp.float32)
    act = up * jax.nn.sigmoid(up)

    B = xb.shape[0]
    maskf = jnp.concatenate(
        [jnp.broadcast_to(ohg[:, e:e + 1], (B, F)) for e in range(E)], axis=1)
    act = act * maskf

    expert_out = jnp.dot(act.astype(jnp.bfloat16), down_bf[...],
                         preferred_element_type=jnp.float32)

    out_ref[...] = xb + expert_out
    ids_ref[0, 0, :] = ids


def kernel(x, up_proj, down_proj, router_w, gate_w):
    N, H = x.shape
    E, _, F = up_proj.shape

    BLK = 512
    grid = N // BLK
    out, ids3 = pl.pallas_call(
        _moe_kernel,
        grid=(grid,),
        in_specs=[
            pl.BlockSpec((BLK, H), lambda i: (i, 0)),
            pl.BlockSpec((E, H, F), lambda i: (0, 0, 0)),
            pl.BlockSpec((E, F, H), lambda i: (0, 0, 0)),
            pl.BlockSpec((E, H), lambda i: (0, 0)),
            pl.BlockSpec((1, H), lambda i: (0, 0)),
        ],
        out_specs=[
            pl.BlockSpec((BLK, H), lambda i: (i, 0)),
            pl.BlockSpec((1, 1, BLK), lambda i: (i, 0, 0)),
        ],
        out_shape=[
            jax.ShapeDtypeStruct((N, H), jnp.float32),
            jax.ShapeDtypeStruct((grid, 1, BLK), jnp.int32),
        ],
        scratch_shapes=[
            pltpu.VMEM((H, E * F), jnp.bfloat16),
            pltpu.VMEM((E * F, H), jnp.bfloat16),
        ],
    )(x, up_proj, down_proj, router_w, gate_w)
    return out, ids3.reshape(N)


# speedup vs baseline: 1.0717x; 1.0220x over previous
import jax
import jax.numpy as jnp
from jax.experimental import pallas as pl
from jax.experimental.pallas import tpu as pltpu


def _moe_kernel(x_ref, up_ref, down_ref, rw_ref, gw_ref, out_ref, ids_ref,
                up_bf, down_bf):
    E, H, F = up_ref.shape

    @pl.when(pl.program_id(0) == 0)
    def _pack_weights():
        for e in range(E):
            up_bf[:, e * F:(e + 1) * F] = up_ref[e].astype(jnp.bfloat16)
            down_bf[e * F:(e + 1) * F, :] = down_ref[e].astype(jnp.bfloat16)

    xb = x_ref[...]
    logits = jax.lax.dot_general(
        xb, rw_ref[...], (((1,), (1,)), ((), ())),
        preferred_element_type=jnp.float32)
    ids = jnp.argmax(logits, axis=-1).astype(jnp.int32)

    xbf = xb.astype(jnp.bfloat16)
    up = jnp.dot(xbf, up_bf[...], preferred_element_type=jnp.float32)
    act = up * jax.nn.sigmoid(up)

    B, EF = act.shape
    col_expert = jax.lax.broadcasted_iota(jnp.int32, (B, EF), 1) // F
    act = jnp.where(col_expert == ids[:, None], act, 0.0)

    expert_out = jnp.dot(act.astype(jnp.bfloat16), down_bf[...],
                         preferred_element_type=jnp.float32)

    gate_logit = jax.lax.dot_general(
        xb, gw_ref[...], (((1,), (1,)), ((), ())),
        preferred_element_type=jnp.float32)
    gate = jax.nn.sigmoid(gate_logit)

    out_ref[...] = xb + gate * expert_out
    ids_ref[0, 0, :] = ids


def kernel(x, up_proj, down_proj, router_w, gate_w):
    N, H = x.shape
    E, _, F = up_proj.shape

    BLK = 512
    grid = N // BLK
    out, ids3 = pl.pallas_call(
        _moe_kernel,
        grid=(grid,),
        in_specs=[
            pl.BlockSpec((BLK, H), lambda i: (i, 0)),
            pl.BlockSpec((E, H, F), lambda i: (0, 0, 0)),
            pl.BlockSpec((E, F, H), lambda i: (0, 0, 0)),
            pl.BlockSpec((E, H), lambda i: (0, 0)),
            pl.BlockSpec((1, H), lambda i: (0, 0)),
        ],
        out_specs=[
            pl.BlockSpec((BLK, H), lambda i: (i, 0)),
            pl.BlockSpec((1, 1, BLK), lambda i: (i, 0, 0)),
        ],
        out_shape=[
            jax.ShapeDtypeStruct((N, H), jnp.float32),
            jax.ShapeDtypeStruct((grid, 1, BLK), jnp.int32),
        ],
        scratch_shapes=[
            pltpu.VMEM((H, E * F), jnp.bfloat16),
            pltpu.VMEM((E * F, H), jnp.bfloat16),
        ],
    )(x, up_proj, down_proj, router_w, gate_w)
    return out, ids3.reshape(N)
